# baseline (speedup 1.0000x reference)
import jax
import jax.numpy as jnp
import numpy as np
from jax.experimental import pallas as pl
from jax.experimental.pallas import tpu as pltpu

N_HEADS = 16
TOPK = 8
C = N_HEADS * TOPK
HP = jax.lax.Precision.HIGHEST


def _mega_kernel(te_ref, ie_ref, tW_ref, tbc_ref, iW_ref, ibc_ref, ggc_ref,
                 gbc_ref, mk_ref, mv_ref, kW_ref, kb_ref, vW_ref, vb_ref,
                 qW_ref, qb_ref, oW_ref, x_ref, ob_ref, ng_ref, nb_ref,
                 out_ref, mt_s, w1_s, b1_s, w2_s):
    b = pl.program_id(0)
    s = pl.program_id(1)
    D = qW_ref.shape[0]
    dh = D // N_HEADS
    B = te_ref.shape[0]
    M = mk_ref.shape[0]

    @pl.when((b == 0) & (s == 0))
    def _prep():
        tW = tW_ref[:]
        iW = iW_ref[:]
        bias_c = tbc_ref[:] + ibc_ref[:]
        gcols = []
        inv_norms = []
        for bb in range(B):
            te_row = te_ref[bb:bb + 1, :]
            ie_row = ie_ref[bb:bb + 1, :]
            gcol = (jnp.sum(tW * te_row, axis=1, keepdims=True)
                    + jnp.sum(iW * ie_row, axis=1, keepdims=True) + bias_c)
            mu = jnp.mean(gcol, axis=0, keepdims=True)
            var = jnp.mean(jnp.square(gcol - mu), axis=0, keepdims=True)
            guide = ((gcol - mu) * jax.lax.rsqrt(var + 1e-5) * ggc_ref[:]
                     + gbc_ref[:])
            nrm = jnp.sqrt(jnp.sum(guide * guide, axis=0, keepdims=True))
            gcols.append(guide)
            inv_norms.append(1.0 / jnp.maximum(nrm, 1e-8))
        gmat = jnp.concatenate(gcols, axis=1)
        ginv = jnp.concatenate(inv_norms, axis=1)
        mk = mk_ref[:]
        knorm = jnp.sqrt(jnp.sum(mk * mk, axis=1, keepdims=True))
        kn = mk / jnp.maximum(knorm, 1e-8)
        sim = jax.lax.dot_general(kn, gmat, (((1,), (0,)), ((), ())),
                                  precision=HP,
                                  preferred_element_type=jnp.float32) * ginv
        iota = jax.lax.broadcasted_iota(jnp.int32, (M, B), 0)
        val = sim
        cols = []
        for t in range(TOPK):
            mx = jnp.max(val, axis=0, keepdims=True)
            cand = jnp.where(val == mx, iota, M)
            sel = jnp.min(cand, axis=0, keepdims=True)
            oh = (iota == sel).astype(jnp.float32)
            cols.append(oh)
            val = jnp.where(iota == sel, -jnp.inf, val)
        ohs = jnp.concatenate(cols, axis=1)
        mt_s[...] = jax.lax.dot_general(ohs, mv_ref[:], (((0,), (0,)), ((), ())),
                                        precision=HP,
                                        preferred_element_type=jnp.float32)

        mt = mt_s[...]
        K = jax.lax.dot_general(mt, kW_ref[:], (((1,), (1,)), ((), ())),
                                preferred_element_type=jnp.float32) + kb_ref[:]
        V = jax.lax.dot_general(mt, vW_ref[:], (((1,), (1,)), ((), ())),
                                preferred_element_type=jnp.float32) + vb_ref[:]
        scale = 1.0 / np.sqrt(dh)
        hc = jax.lax.broadcasted_iota(jnp.int32, (C, D), 0) // TOPK
        hd = jax.lax.broadcasted_iota(jnp.int32, (C, D), 1) // dh
        Mmask = (hc == hd).astype(jnp.float32)
        ci = jax.lax.broadcasted_iota(jnp.int32, (C, TOPK * B), 0)
        rj = jax.lax.broadcasted_iota(jnp.int32, (C, TOPK * B), 1)
        for bb in range(B):
            P = (rj == (ci % TOPK) * B + bb).astype(jnp.float32)
            KbM = jnp.dot(P, K, preferred_element_type=jnp.float32) * Mmask
            VbM = jnp.dot(P, V, preferred_element_type=jnp.float32) * Mmask
            w1_s[bb] = scale * jax.lax.dot_general(
                qW_ref[:], KbM, (((0,), (1,)), ((), ())),
                preferred_element_type=jnp.float32)
            b1_s[bb] = scale * jax.lax.dot_general(
                qb_ref[:], KbM, (((1,), (1,)), ((), ())),
                preferred_element_type=jnp.float32)
            w2_s[bb] = jax.lax.dot_general(
                VbM, oW_ref[:], (((1,), (1,)), ((), ())),
                preferred_element_type=jnp.float32)

    xt = x_ref[0]
    l = (jnp.dot(xt, w1_s[b], preferred_element_type=jnp.float32) + b1_s[b])
    mx = jnp.max(l, axis=1, keepdims=True)
    e = jnp.exp(l - mx)
    gi = jax.lax.broadcasted_iota(jnp.int32, (C, C), 0) // TOPK
    gj = jax.lax.broadcasted_iota(jnp.int32, (C, C), 1) // TOPK
    G = (gi == gj).astype(jnp.float32)
    sums = jnp.dot(e, G, preferred_element_type=jnp.float32)
    p = e / sums
    y = jnp.dot(p, w2_s[b], preferred_element_type=jnp.float32) + ob_ref[:]
    r = xt + y
    mu = jnp.mean(r, axis=1, keepdims=True)
    var = jnp.mean(jnp.square(r - mu), axis=1, keepdims=True)
    out_ref[0] = (r - mu) * jax.lax.rsqrt(var + 1e-5) * ng_ref[:] + nb_ref[:]


def kernel(x, mem_keys, mem_values, text_emb, image_emb, text_W, text_b,
           img_W, img_b, gn_g, gn_b, qW, qb, kW, kb, vW, vb, oW, ob, n_g, n_b):
    B, S, D = x.shape
    M = mem_keys.shape[0]
    TD = text_W.shape[1]
    tbc = text_b.reshape(-1, 1)
    ibc = img_b.reshape(-1, 1)
    ggc = gn_g.reshape(-1, 1)
    gbc = gn_b.reshape(-1, 1)
    qb2 = qb.reshape(1, -1)
    kb2 = kb.reshape(1, -1)
    vb2 = vb.reshape(1, -1)
    ob2 = ob.reshape(1, -1)
    ng2 = n_g.reshape(1, -1)
    nb2 = n_b.reshape(1, -1)

    TS = 512
    full = lambda *shape: pl.BlockSpec(shape, lambda b, s: (0,) * len(shape))
    out = pl.pallas_call(
        _mega_kernel,
        grid=(B, S // TS),
        in_specs=[
            full(B, TD), full(B, D), full(D, TD), full(D, 1), full(D, D),
            full(D, 1), full(D, 1), full(D, 1), full(M, D), full(M, D),
            full(D, D), full(1, D), full(D, D), full(1, D), full(D, D),
            full(1, D), full(D, D),
            pl.BlockSpec((1, TS, D), lambda b, s: (b, s, 0)),
            full(1, D), full(1, D), full(1, D),
        ],
        out_specs=pl.BlockSpec((1, TS, D), lambda b, s: (b, s, 0)),
        out_shape=jax.ShapeDtypeStruct((B, S, D), jnp.float32),
        scratch_shapes=[
            pltpu.VMEM((TOPK * B, D), jnp.float32),
            pltpu.VMEM((B, D, C), jnp.float32),
            pltpu.VMEM((B, 1, C), jnp.float32),
            pltpu.VMEM((B, C, D), jnp.float32),
        ],
    )(text_emb, image_emb, text_W, tbc, img_W, ibc, ggc, gbc, mem_keys,
      mem_values, kW, kb2, vW, vb2, qW, qb2, oW, x, ob2, ng2, nb2)
    return out


# speedup vs baseline: 1.0904x; 1.0904x over previous
import jax
import jax.numpy as jnp
import numpy as np
from jax.experimental import pallas as pl
from jax.experimental.pallas import tpu as pltpu

N_HEADS = 16
TOPK = 8
C = N_HEADS * TOPK
HP = jax.lax.Precision.HIGHEST


def _mega_kernel(te_ref, ie_ref, tW_ref, tbc_ref, iW_ref, ibc_ref, ggc_ref,
                 gbc_ref, mk_ref, mv_ref, kW_ref, kb_ref, vW_ref, vb_ref,
                 qW_ref, qb_ref, oW_ref, x_ref, ob_ref, ng_ref, nb_ref,
                 out_ref, mt_s, w1_s, b1_s, w2_s, kW_v, vW_v, qW_v, oW_v,
                 sem_k, sem_v, sem_q, sem_o):
    b = pl.program_id(0)
    s = pl.program_id(1)
    D = qW_v.shape[0]
    dh = D // N_HEADS
    B = te_ref.shape[0]
    M = mk_ref.shape[0]

    @pl.when((b == 0) & (s == 0))
    def _prep():
        cp_k = pltpu.make_async_copy(kW_ref, kW_v, sem_k)
        cp_v = pltpu.make_async_copy(vW_ref, vW_v, sem_v)
        cp_q = pltpu.make_async_copy(qW_ref, qW_v, sem_q)
        cp_o = pltpu.make_async_copy(oW_ref, oW_v, sem_o)
        cp_k.start()
        cp_v.start()
        cp_q.start()
        cp_o.start()
        tW = tW_ref[:]
        iW = iW_ref[:]
        bias_c = tbc_ref[:] + ibc_ref[:]
        gcols = []
        inv_norms = []
        for bb in range(B):
            te_row = te_ref[bb:bb + 1, :]
            ie_row = ie_ref[bb:bb + 1, :]
            gcol = (jnp.sum(tW * te_row, axis=1, keepdims=True)
                    + jnp.sum(iW * ie_row, axis=1, keepdims=True) + bias_c)
            mu = jnp.mean(gcol, axis=0, keepdims=True)
            var = jnp.mean(jnp.square(gcol - mu), axis=0, keepdims=True)
            guide = ((gcol - mu) * jax.lax.rsqrt(var + 1e-5) * ggc_ref[:]
                     + gbc_ref[:])
            nrm = jnp.sqrt(jnp.sum(guide * guide, axis=0, keepdims=True))
            gcols.append(guide)
            inv_norms.append(1.0 / jnp.maximum(nrm, 1e-8))
        gmat = jnp.concatenate(gcols, axis=1)
        ginv = jnp.concatenate(inv_norms, axis=1)
        mk = mk_ref[:]
        knorm = jnp.sqrt(jnp.sum(mk * mk, axis=1, keepdims=True))
        kn = mk / jnp.maximum(knorm, 1e-8)
        sim = jax.lax.dot_general(kn, gmat, (((1,), (0,)), ((), ())),
                                  precision=HP,
                                  preferred_element_type=jnp.float32) * ginv
        iota = jax.lax.broadcasted_iota(jnp.int32, (M, B), 0)
        val = sim
        cols = []
        for t in range(TOPK):
            mx = jnp.max(val, axis=0, keepdims=True)
            cand = jnp.where(val == mx, iota, M)
            sel = jnp.min(cand, axis=0, keepdims=True)
            oh = (iota == sel).astype(jnp.float32)
            cols.append(oh)
            val = jnp.where(iota == sel, -jnp.inf, val)
        ohs = jnp.concatenate(cols, axis=1)
        mt_s[...] = jax.lax.dot_general(ohs, mv_ref[:], (((0,), (0,)), ((), ())),
                                        precision=HP,
                                        preferred_element_type=jnp.float32)

        mt = mt_s[...]
        cp_k.wait()
        K = jax.lax.dot_general(mt, kW_v[:], (((1,), (1,)), ((), ())),
                                preferred_element_type=jnp.float32) + kb_ref[:]
        cp_v.wait()
        V = jax.lax.dot_general(mt, vW_v[:], (((1,), (1,)), ((), ())),
                                preferred_element_type=jnp.float32) + vb_ref[:]
        cp_q.wait()
        cp_o.wait()
        scale = 1.0 / np.sqrt(dh)
        hc = jax.lax.broadcasted_iota(jnp.int32, (C, D), 0) // TOPK
        hd = jax.lax.broadcasted_iota(jnp.int32, (C, D), 1) // dh
        Mmask = (hc == hd).astype(jnp.float32)
        ci = jax.lax.broadcasted_iota(jnp.int32, (C, TOPK * B), 0)
        rj = jax.lax.broadcasted_iota(jnp.int32, (C, TOPK * B), 1)
        for bb in range(B):
            P = (rj == (ci % TOPK) * B + bb).astype(jnp.float32)
            KbM = jnp.dot(P, K, preferred_element_type=jnp.float32) * Mmask
            VbM = jnp.dot(P, V, preferred_element_type=jnp.float32) * Mmask
            w1_s[bb] = scale * jax.lax.dot_general(
                qW_v[:], KbM, (((0,), (1,)), ((), ())),
                preferred_element_type=jnp.float32)
            b1_s[bb] = scale * jax.lax.dot_general(
                qb_ref[:], KbM, (((1,), (1,)), ((), ())),
                preferred_element_type=jnp.float32)
            w2_s[bb] = jax.lax.dot_general(
                VbM, oW_v[:], (((1,), (1,)), ((), ())),
                preferred_element_type=jnp.float32)

    xt = x_ref[0]
    l = (jnp.dot(xt, w1_s[b], preferred_element_type=jnp.float32) + b1_s[b])
    mx = jnp.max(l, axis=1, keepdims=True)
    e = jnp.exp(l - mx)
    gi = jax.lax.broadcasted_iota(jnp.int32, (C, C), 0) // TOPK
    gj = jax.lax.broadcasted_iota(jnp.int32, (C, C), 1) // TOPK
    G = (gi == gj).astype(jnp.float32)
    sums = jnp.dot(e, G, preferred_element_type=jnp.float32)
    p = e / sums
    y = jnp.dot(p, w2_s[b], preferred_element_type=jnp.float32) + ob_ref[:]
    r = xt + y
    mu = jnp.mean(r, axis=1, keepdims=True)
    var = jnp.mean(jnp.square(r - mu), axis=1, keepdims=True)
    out_ref[0] = (r - mu) * jax.lax.rsqrt(var + 1e-5) * ng_ref[:] + nb_ref[:]


def kernel(x, mem_keys, mem_values, text_emb, image_emb, text_W, text_b,
           img_W, img_b, gn_g, gn_b, qW, qb, kW, kb, vW, vb, oW, ob, n_g, n_b):
    B, S, D = x.shape
    M = mem_keys.shape[0]
    TD = text_W.shape[1]
    tbc = text_b.reshape(-1, 1)
    ibc = img_b.reshape(-1, 1)
    ggc = gn_g.reshape(-1, 1)
    gbc = gn_b.reshape(-1, 1)
    qb2 = qb.reshape(1, -1)
    kb2 = kb.reshape(1, -1)
    vb2 = vb.reshape(1, -1)
    ob2 = ob.reshape(1, -1)
    ng2 = n_g.reshape(1, -1)
    nb2 = n_b.reshape(1, -1)

    TS = 512
    full = lambda *shape: pl.BlockSpec(shape, lambda b, s: (0,) * len(shape))
    out = pl.pallas_call(
        _mega_kernel,
        grid=(B, S // TS),
        in_specs=[
            full(B, TD), full(B, D), full(D, TD), full(D, 1), full(D, D),
            full(D, 1), full(D, 1), full(D, 1), full(M, D), full(M, D),
            pl.BlockSpec(memory_space=pltpu.MemorySpace.HBM), full(1, D),
            pl.BlockSpec(memory_space=pltpu.MemorySpace.HBM), full(1, D),
            pl.BlockSpec(memory_space=pltpu.MemorySpace.HBM), full(1, D),
            pl.BlockSpec(memory_space=pltpu.MemorySpace.HBM),
            pl.BlockSpec((1, TS, D), lambda b, s: (b, s, 0)),
            full(1, D), full(1, D), full(1, D),
        ],
        out_specs=pl.BlockSpec((1, TS, D), lambda b, s: (b, s, 0)),
        out_shape=jax.ShapeDtypeStruct((B, S, D), jnp.float32),
        scratch_shapes=[
            pltpu.VMEM((TOPK * B, D), jnp.float32),
            pltpu.VMEM((B, D, C), jnp.float32),
            pltpu.VMEM((B, 1, C), jnp.float32),
            pltpu.VMEM((B, C, D), jnp.float32),
            pltpu.VMEM((D, D), jnp.float32),
            pltpu.VMEM((D, D), jnp.float32),
            pltpu.VMEM((D, D), jnp.float32),
            pltpu.VMEM((D, D), jnp.float32),
            pltpu.SemaphoreType.DMA,
            pltpu.SemaphoreType.DMA,
            pltpu.SemaphoreType.DMA,
            pltpu.SemaphoreType.DMA,
        ],
    )(text_emb, image_emb, text_W, tbc, img_W, ibc, ggc, gbc, mem_keys,
      mem_values, kW, kb2, vW, vb2, qW, qb2, oW, x, ob2, ng2, nb2)
    return out
